# Initial kernel scaffold; baseline (speedup 1.0000x reference)
import jax
import jax.numpy as jnp
from jax.experimental import pallas as pl


def kernel(x, table):
    raise NotImplementedError("write your pallas kernel here")


# speedup vs baseline: 2.5465x; 2.5465x over previous
import functools

import jax
import jax.numpy as jnp
from jax import lax
from jax.experimental import pallas as pl
from jax.experimental.pallas import tpu as pltpu
from jax.experimental.pallas import tpu_sc as plsc

EMBED_D = 32
NC = 2
NS = 16
NW = NC * NS
CH = 128


@functools.partial(jax.jit, static_argnames=())
def _sc_gather(idx_flat, table):
    n = idx_flat.shape[0]
    b_per_w = n // NW
    n_ch = b_per_w // CH
    mesh = plsc.VectorSubcoreMesh(core_axis_name="c", subcore_axis_name="s")

    @functools.partial(
        pl.kernel,
        mesh=mesh,
        out_type=jax.ShapeDtypeStruct((n, EMBED_D), jnp.float32),
        scratch_types=[
            pltpu.VMEM((CH,), jnp.int32),
            pltpu.VMEM((CH, EMBED_D), jnp.float32),
            pltpu.SemaphoreType.DMA,
        ],
        compiler_params=pltpu.CompilerParams(use_tc_tiling_on_sc=False),
    )
    def k(table_hbm, idx_hbm, out_hbm, idx_v, rows_v, sem):
        wid = lax.axis_index("s") * NC + lax.axis_index("c")
        base = wid * b_per_w

        def body(g, carry):
            off = base + g * CH
            pltpu.sync_copy(idx_hbm.at[pl.ds(off, CH)], idx_v)
            pltpu.async_copy(table_hbm.at[idx_v], rows_v, sem).wait()
            pltpu.sync_copy(rows_v, out_hbm.at[pl.ds(off, CH)])
            return carry

        lax.fori_loop(0, n_ch, body, 0)

    return k(table, idx_flat)


def kernel(x, table):
    B, P, L = x.shape
    out = _sc_gather(x.reshape(-1), table)
    return out.reshape(B, P, L, EMBED_D)


# speedup vs baseline: 3.0122x; 1.1828x over previous
import functools

import jax
import jax.numpy as jnp
from jax import lax
from jax.experimental import pallas as pl
from jax.experimental.pallas import tpu as pltpu
from jax.experimental.pallas import tpu_sc as plsc

EMBED_D = 32
NC = 2
NS = 16
NW = NC * NS
CH = 1600


@jax.jit
def _sc_gather(idx_flat, table):
    n = idx_flat.shape[0]
    b_per_w = n // NW
    n_ch = b_per_w // CH
    n_pairs = n_ch // 2
    mesh = plsc.VectorSubcoreMesh(core_axis_name="c", subcore_axis_name="s")

    @functools.partial(
        pl.kernel,
        mesh=mesh,
        out_type=jax.ShapeDtypeStruct((n, EMBED_D), jnp.float32),
        scratch_types=[
            pltpu.VMEM((CH,), jnp.int32),
            pltpu.VMEM((CH,), jnp.int32),
            pltpu.VMEM((CH, EMBED_D), jnp.float32),
            pltpu.VMEM((CH, EMBED_D), jnp.float32),
            pltpu.SemaphoreType.DMA,
            pltpu.SemaphoreType.DMA,
        ],
        compiler_params=pltpu.CompilerParams(use_tc_tiling_on_sc=False),
    )
    def k(table_hbm, idx_hbm, out_hbm, idx0, idx1, rows0, rows1, sem0, sem1):
        wid = lax.axis_index("s") * NC + lax.axis_index("c")
        base = wid * b_per_w

        pltpu.sync_copy(idx_hbm.at[pl.ds(base, CH)], idx0)
        pltpu.async_copy(table_hbm.at[idx0], rows0, sem0)

        def body(h, carry):
            g0 = 2 * h
            off0 = base + g0 * CH
            off1 = off0 + CH

            pltpu.sync_copy(idx_hbm.at[pl.ds(off1, CH)], idx1)
            pltpu.async_copy(table_hbm.at[idx1], rows1, sem1)

            pltpu.make_async_copy(table_hbm.at[idx0], rows0, sem0).wait()
            pltpu.sync_copy(rows0, out_hbm.at[pl.ds(off0, CH)])

            @pl.when(h < n_pairs - 1)
            def _():
                off2 = off1 + CH
                pltpu.sync_copy(idx_hbm.at[pl.ds(off2, CH)], idx0)
                pltpu.async_copy(table_hbm.at[idx0], rows0, sem0)

            pltpu.make_async_copy(table_hbm.at[idx1], rows1, sem1).wait()
            pltpu.sync_copy(rows1, out_hbm.at[pl.ds(off1, CH)])
            return carry

        lax.fori_loop(0, n_pairs, body, 0)

    return k(table, idx_flat)


def kernel(x, table):
    B, P, L = x.shape
    out = _sc_gather(x.reshape(-1), table)
    return out.reshape(B, P, L, EMBED_D)
